# baseline (speedup 1.0000x reference)
import math
import jax
import jax.numpy as jnp
from jax.experimental import pallas as pl
from jax.experimental.pallas import tpu as pltpu

N = 2048
M = 1024
D = 1024
H = 16
DH = D // H
TOP_N = 10

_BN = 512


def _mm_t(a, b):
    return jax.lax.dot_general(
        a, b, (((1,), (1,)), ((), ())), preferred_element_type=jnp.float32)


def _layernorm(x, g, b):
    mu = jnp.mean(x, axis=-1, keepdims=True)
    var = jnp.mean((x - mu) ** 2, axis=-1, keepdims=True)
    return (x - mu) * jax.lax.rsqrt(var + 1e-5) * g + b


def _kv_kernel(ep_ref, em_ref, wk_ref, bk_ref, wv_ref, bv_ref,
               k_ref, v_ref):
    kf = _mm_t(ep_ref[...], wk_ref[...]) + bk_ref[...]
    vf = _mm_t(em_ref[...], wv_ref[...]) + bv_ref[...]
    for hh in range(H):
        k_ref[hh] = kf[:, hh * DH:(hh + 1) * DH]
        v_ref[hh] = vf[:, hh * DH:(hh + 1) * DH]


def _kv(ep, em, Wk, bk, Wv, bv):
    row = lambda t: t.reshape(1, D)
    bm = 512
    full = pl.BlockSpec((D, D), lambda i: (0, 0))
    vec = pl.BlockSpec((1, D), lambda i: (0, 0))
    return pl.pallas_call(
        _kv_kernel,
        grid=(M // bm,),
        in_specs=[
            pl.BlockSpec((bm, D), lambda i: (i, 0)),
            pl.BlockSpec((bm, D), lambda i: (i, 0)),
            full, vec, full, vec,
        ],
        out_specs=[
            pl.BlockSpec((H, bm, DH), lambda i: (0, i, 0)),
            pl.BlockSpec((H, bm, DH), lambda i: (0, i, 0)),
        ],
        out_shape=[
            jax.ShapeDtypeStruct((H, M, DH), jnp.float32),
            jax.ShapeDtypeStruct((H, M, DH), jnp.float32),
        ],
    )(ep, em, Wk, row(bk), Wv, row(bv))


def _topn_probs(s):
    bn = s.shape[0]
    neg = jnp.float32(-jnp.inf)
    m = jnp.max(s, axis=1, keepdims=True)
    rowmax = m
    for _ in range(TOP_N - 1):
        m = jnp.max(jnp.where(s < m, s, neg), axis=1, keepdims=True)
    m10 = m
    keep = s >= m10
    k10 = jnp.sum(jnp.where(keep, 1.0, 0.0), axis=1, keepdims=True)
    e = jnp.exp(s - rowmax)

    def _fast(_):
        return jnp.where(keep, e, 0.0)

    def _exact(_):
        cnt = jnp.zeros((bn, 1), jnp.float32)
        thr = jnp.full((bn, 1), neg, jnp.float32)
        t_m = s
        for _ in range(TOP_N):
            mm = jnp.max(t_m, axis=1, keepdims=True)
            mask = t_m == mm
            c = jnp.sum(jnp.where(mask, 1.0, 0.0), axis=1, keepdims=True)
            t_m = jnp.where(mask, neg, t_m)
            active = cnt < TOP_N
            thr = jnp.where(active, mm, thr)
            cnt = cnt + jnp.where(active, c, 0.0)
        return jnp.where(s >= thr, e, 0.0)

    return jax.lax.cond(jnp.any(k10 > jnp.float32(TOP_N)), _exact, _fast, 0)


def _attn_kernel(x_ref, kh_ref, vh_ref, wq_ref, bq_ref, wo_ref, bo_ref,
                 w1_ref, b1_ref, w2_ref, b2_ref, g1_ref, be1_ref,
                 g2_ref, be2_ref, o_ref, q_s, ao_s):
    h = pl.program_id(1)

    @pl.when(h == 0)
    def _init_q():
        qf = (_mm_t(x_ref[...], wq_ref[...]) + bq_ref[...]) * (
            1.0 / math.sqrt(DH))
        for hh in range(H):
            q_s[hh] = qf[:, hh * DH:(hh + 1) * DH]

    outs = []
    for sub in range(2):
        hh = 2 * h + sub
        q = q_s[hh]
        k = kh_ref[hh]
        v = vh_ref[hh]
        s = _mm_t(q, k)
        p = _topn_probs(s)
        denom = jnp.sum(p, axis=1, keepdims=True)
        o = jnp.dot(p, v, preferred_element_type=jnp.float32)
        outs.append(o / denom)
    ao_s[:, pl.ds(h * 2 * DH, 2 * DH)] = jnp.concatenate(outs, axis=1)

    @pl.when(h == H // 2 - 1)
    def _tail():
        x = x_ref[...]
        z = _mm_t(ao_s[...], wo_ref[...]) + bo_ref[...]
        x1 = _layernorm(x + z, g1_ref[...], be1_ref[...])
        h1 = _mm_t(x1, w1_ref[...]) + b1_ref[...]
        h1 = jnp.where(h1 >= 0.0, h1, 0.01 * h1)
        ff = _mm_t(h1, w2_ref[...]) + b2_ref[...]
        o_ref[...] = _layernorm(x1 + ff, g2_ref[...], be2_ref[...])


def _attention(x, kh, vh, Wq, bq, Wo, bo, W1, b1, W2, b2, g1, be1, g2,
               be2):
    row = lambda t: t.reshape(1, D)
    full = pl.BlockSpec((D, D), lambda i, h: (0, 0))
    vec = pl.BlockSpec((1, D), lambda i, h: (0, 0))
    return pl.pallas_call(
        _attn_kernel,
        grid=(N // _BN, H // 2),
        in_specs=[
            pl.BlockSpec((_BN, D), lambda i, h: (i, 0)),
            pl.BlockSpec((H, M, DH), lambda i, h: (0, 0, 0)),
            pl.BlockSpec((H, M, DH), lambda i, h: (0, 0, 0)),
            full, vec, full, vec, full, vec, full, vec,
            vec, vec, vec, vec,
        ],
        out_specs=pl.BlockSpec((_BN, D), lambda i, h: (i, 0)),
        out_shape=jax.ShapeDtypeStruct((N, D), jnp.float32),
        scratch_shapes=[
            pltpu.VMEM((H, _BN, DH), jnp.float32),
            pltpu.VMEM((_BN, D), jnp.float32),
        ],
    )(x, kh, vh, Wq, row(bq), Wo, row(bo), W1, row(b1), W2, row(b2),
      row(g1), row(be1), row(g2), row(be2))


def kernel(visit_rep, E_mem_patient_rep, E_mem_med_rep, Wq, bq, Wk, bk,
           Wv, bv, Wo, bo, W1, b1, W2, b2, g1, be1, g2, be2):
    kh, vh = _kv(E_mem_patient_rep, E_mem_med_rep, Wk, bk, Wv, bv)
    return _attention(visit_rep, kh, vh, Wq, bq, Wo, bo, W1, b1,
                      W2, b2, g1, be1, g2, be2)


# speedup vs baseline: 1.1974x; 1.1974x over previous
import math
import jax
import jax.numpy as jnp
from jax.experimental import pallas as pl
from jax.experimental.pallas import tpu as pltpu

N = 2048
M = 1024
D = 1024
H = 16
DH = D // H
TOP_N = 10

_BN = 512


def _mm_t(a, b):
    return jax.lax.dot_general(
        a, b, (((1,), (1,)), ((), ())), preferred_element_type=jnp.float32)


def _layernorm(x, g, b):
    mu = jnp.mean(x, axis=-1, keepdims=True)
    var = jnp.mean((x - mu) ** 2, axis=-1, keepdims=True)
    return (x - mu) * jax.lax.rsqrt(var + 1e-5) * g + b


def _kv_kernel(ep_ref, em_ref, wk_ref, bk_ref, wv_ref, bv_ref,
               k_ref, v_ref):
    kf = _mm_t(ep_ref[...], wk_ref[...]) + bk_ref[...]
    vf = _mm_t(em_ref[...], wv_ref[...]) + bv_ref[...]
    for hh in range(H):
        k_ref[hh] = kf[:, hh * DH:(hh + 1) * DH]
        v_ref[hh] = vf[:, hh * DH:(hh + 1) * DH]


def _kv(ep, em, Wk, bk, Wv, bv):
    row = lambda t: t.reshape(1, D)
    bm = 512
    full = pl.BlockSpec((D, D), lambda i: (0, 0))
    vec = pl.BlockSpec((1, D), lambda i: (0, 0))
    return pl.pallas_call(
        _kv_kernel,
        grid=(M // bm,),
        in_specs=[
            pl.BlockSpec((bm, D), lambda i: (i, 0)),
            pl.BlockSpec((bm, D), lambda i: (i, 0)),
            full, vec, full, vec,
        ],
        out_specs=[
            pl.BlockSpec((H, bm, DH), lambda i: (0, i, 0)),
            pl.BlockSpec((H, bm, DH), lambda i: (0, i, 0)),
        ],
        out_shape=[
            jax.ShapeDtypeStruct((H, M, DH), jnp.float32),
            jax.ShapeDtypeStruct((H, M, DH), jnp.float32),
        ],
    )(ep, em, Wk, row(bk), Wv, row(bv))


def _topn_threshold(s):
    bn = s.shape[0]
    neg = jnp.float32(-jnp.inf)
    m = jnp.max(s, axis=1, keepdims=True)
    rowmax = m
    for _ in range(TOP_N - 1):
        m = jnp.max(jnp.where(s < m, s, neg), axis=1, keepdims=True)
    m10 = m
    k10 = jnp.sum(jnp.where(s >= m10, 1.0, 0.0), axis=1, keepdims=True)

    def _fast(_):
        return m10

    def _exact(_):
        cnt = jnp.zeros((bn, 1), jnp.float32)
        thr = jnp.full((bn, 1), neg, jnp.float32)
        t_m = s
        for _ in range(TOP_N):
            mm = jnp.max(t_m, axis=1, keepdims=True)
            mask = t_m == mm
            c = jnp.sum(jnp.where(mask, 1.0, 0.0), axis=1, keepdims=True)
            t_m = jnp.where(mask, neg, t_m)
            active = cnt < TOP_N
            thr = jnp.where(active, mm, thr)
            cnt = cnt + jnp.where(active, c, 0.0)
        return thr

    thr = jax.lax.cond(jnp.any(k10 > jnp.float32(TOP_N)), _exact, _fast, 0)
    return thr, rowmax


def _attn_kernel(x_ref, kh_ref, vh_ref, wq_ref, bq_ref, wo_ref, bo_ref,
                 w1_ref, b1_ref, w2_ref, b2_ref, g1_ref, be1_ref,
                 g2_ref, be2_ref, o_ref, q_s, ao_s):
    h = pl.program_id(1)

    @pl.when(h == 0)
    def _init_q():
        qf = (_mm_t(x_ref[...], wq_ref[...]) + bq_ref[...]) * (
            1.0 / math.sqrt(DH))
        for hh in range(H):
            q_s[hh] = qf[:, hh * DH:(hh + 1) * DH]

    outs = []
    for sub in range(2):
        hh = 2 * h + sub
        q = q_s[hh]
        k = kh_ref[hh]
        v = vh_ref[hh]
        s = _mm_t(q, k)
        thr, rowmax = _topn_threshold(s)
        p = jnp.where(s >= thr, jnp.exp(s - rowmax), 0.0)
        denom = jnp.sum(p, axis=1, keepdims=True)
        o = jnp.dot(p, v, preferred_element_type=jnp.float32)
        outs.append(o / denom)
    ao_s[:, pl.ds(h * 2 * DH, 2 * DH)] = jnp.concatenate(outs, axis=1)

    @pl.when(h == H // 2 - 1)
    def _tail():
        x = x_ref[...]
        z = _mm_t(ao_s[...], wo_ref[...]) + bo_ref[...]
        x1 = _layernorm(x + z, g1_ref[...], be1_ref[...])
        h1 = _mm_t(x1, w1_ref[...]) + b1_ref[...]
        h1 = jnp.where(h1 >= 0.0, h1, 0.01 * h1)
        ff = _mm_t(h1, w2_ref[...]) + b2_ref[...]
        o_ref[...] = _layernorm(x1 + ff, g2_ref[...], be2_ref[...])


def _attention(x, kh, vh, Wq, bq, Wo, bo, W1, b1, W2, b2, g1, be1, g2,
               be2):
    row = lambda t: t.reshape(1, D)
    full = pl.BlockSpec((D, D), lambda i, h: (0, 0))
    vec = pl.BlockSpec((1, D), lambda i, h: (0, 0))
    return pl.pallas_call(
        _attn_kernel,
        grid=(N // _BN, H // 2),
        in_specs=[
            pl.BlockSpec((_BN, D), lambda i, h: (i, 0)),
            pl.BlockSpec((H, M, DH), lambda i, h: (0, 0, 0)),
            pl.BlockSpec((H, M, DH), lambda i, h: (0, 0, 0)),
            full, vec, full, vec, full, vec, full, vec,
            vec, vec, vec, vec,
        ],
        out_specs=pl.BlockSpec((_BN, D), lambda i, h: (i, 0)),
        out_shape=jax.ShapeDtypeStruct((N, D), jnp.float32),
        scratch_shapes=[
            pltpu.VMEM((H, _BN, DH), jnp.float32),
            pltpu.VMEM((_BN, D), jnp.float32),
        ],
    )(x, kh, vh, Wq, row(bq), Wo, row(bo), W1, row(b1), W2, row(b2),
      row(g1), row(be1), row(g2), row(be2))


def kernel(visit_rep, E_mem_patient_rep, E_mem_med_rep, Wq, bq, Wk, bk,
           Wv, bv, Wo, bo, W1, b1, W2, b2, g1, be1, g2, be2):
    kh, vh = _kv(E_mem_patient_rep, E_mem_med_rep, Wk, bk, Wv, bv)
    return _attention(visit_rep, kh, vh, Wq, bq, Wo, bo, W1, b1,
                      W2, b2, g1, be1, g2, be2)
